# baseline (speedup 1.0000x reference)
import functools

import jax
import jax.numpy as jnp
from jax import lax
from jax.experimental import pallas as pl
from jax.experimental.pallas import tpu as pltpu
from jax.experimental.pallas import tpu_sc as plsc

_ROWS = 64
_COLS = 32768
_N = _ROWS * _COLS
_NC = 2
_NS = 16
_NW = _NC * _NS
_PER_W = _N // _NW
_HALF = _PER_W // 2
_LANES = 16
_UNROLL = 8
_TOP_K = 64
_SEL_THR = 0.1


_ROWS_W = 8
_COLS_W = _COLS // 4
_COLS_H = _COLS_W // 2


def _sc_partials(attn):
    mesh = plsc.VectorSubcoreMesh(core_axis_name="c", subcore_axis_name="s")

    @functools.partial(
        pl.kernel,
        mesh=mesh,
        out_type=(
            jax.ShapeDtypeStruct((3 * _NW * _LANES,), jnp.float32),
            jax.ShapeDtypeStruct((_ROWS, _COLS), jnp.float32),
        ),
        compiler_params=pltpu.CompilerParams(
            use_tc_tiling_on_sc=True, needs_layout_passes=False),
        scratch_types=[
            pltpu.VMEM((2, _ROWS_W, _COLS_H), jnp.float32),
            pltpu.VMEM((3, _LANES), jnp.float32),
            pltpu.SemaphoreType.DMA,
            pltpu.SemaphoreType.DMA,
            pltpu.SemaphoreType.DMA,
            pltpu.SemaphoreType.DMA,
        ],
    )
    def k(x_hbm, out_hbm, attn_hbm, buf, stage, sem0, sem1, sem2, sem3):
        wid = lax.axis_index("s") * _NC + lax.axis_index("c")
        r0 = (wid // 4) * _ROWS_W
        c0 = (wid % 4) * _COLS_W

        cp0 = pltpu.async_copy(
            x_hbm.at[pl.ds(r0, _ROWS_W), pl.ds(c0, _COLS_H)], buf.at[0], sem0)
        cp1 = pltpu.async_copy(
            x_hbm.at[pl.ds(r0, _ROWS_W), pl.ds(c0 + _COLS_H, _COLS_H)],
            buf.at[1], sem1)

        fzeros = jnp.zeros((_LANES,), jnp.float32)
        izeros = jnp.zeros((_LANES,), jnp.int32)
        init = (fzeros, fzeros, izeros) * _UNROLL
        group = _LANES * _UNROLL

        def make_body(h, r):
            def body(i, carry):
                accs = list(carry)
                for j in range(_UNROLL):
                    v = buf[h, r, pl.ds(i + j * _LANES, _LANES)]
                    accs[3 * j] = accs[3 * j] + v
                    accs[3 * j + 1] = accs[3 * j + 1] + v * v
                    accs[3 * j + 2] = accs[3 * j + 2] + (
                        plsc.all_reduce_population_count(v > _SEL_THR))
                return tuple(accs)
            return body

        cp0.wait()
        wr0 = pltpu.async_copy(
            buf.at[0], attn_hbm.at[pl.ds(r0, _ROWS_W), pl.ds(c0, _COLS_H)],
            sem2)
        res = init
        for r in range(_ROWS_W):
            res = plsc.parallel_loop(
                0, _COLS_H, step=group, unroll=2, carry=res)(make_body(0, r))
        cp1.wait()
        wr1 = pltpu.async_copy(
            buf.at[1],
            attn_hbm.at[pl.ds(r0, _ROWS_W), pl.ds(c0 + _COLS_H, _COLS_H)],
            sem3)
        for r in range(_ROWS_W):
            res = plsc.parallel_loop(
                0, _COLS_H, step=group, unroll=2, carry=res)(make_body(1, r))
        wr0.wait()
        wr1.wait()

        def tree_sum(vals):
            vals = list(vals)
            while len(vals) > 1:
                vals = [a + b for a, b in zip(vals[::2], vals[1::2])]
            return vals[0]

        s = tree_sum(res[0::3])
        q = tree_sum(res[1::3])
        c = tree_sum(res[2::3])
        stage[0] = s
        stage[1] = q
        stage[2] = c.astype(jnp.float32) * (1.0 / _LANES)
        for j in range(3):
            pltpu.sync_copy(
                stage.at[j],
                out_hbm.at[pl.ds((j * _NW + wid) * _LANES, _LANES)])

    return k(attn)


def _finalize(partials):
    def body(p_ref, inten_ref, qual_ref, topk_ref, spars_ref):
        w = _NW * _LANES
        s = jnp.sum(p_ref[pl.ds(0, w)])
        sq = jnp.sum(p_ref[pl.ds(w, w)])
        c = jnp.sum(p_ref[pl.ds(2 * w, w)])
        n = jnp.float32(_N)
        mean = s / n
        var = (sq - n * mean * mean) / (n - 1.0)
        std = jnp.sqrt(jnp.maximum(var, 0.0))
        consistency = 1.0 / (1.0 + std)
        focus = jnp.maximum(0.0, mean - 0.5) * 2.0
        inten_ref[0] = mean
        qual_ref[0] = jnp.clip(consistency - focus * 0.1, 0.0, 1.0)
        topk_ref[0] = jnp.minimum(c, jnp.float32(_TOP_K)).astype(jnp.int32)
        spars_ref[0] = 1.0 - c / n

    return pl.pallas_call(
        body,
        out_shape=(
            jax.ShapeDtypeStruct((1,), jnp.float32),
            jax.ShapeDtypeStruct((1,), jnp.float32),
            jax.ShapeDtypeStruct((1,), jnp.int32),
            jax.ShapeDtypeStruct((1,), jnp.float32),
        ),
        out_specs=(pl.BlockSpec(memory_space=pltpu.SMEM),) * 4,
    )(partials)


def kernel(attention_scores, features):
    partials, attn_out = _sc_partials(attention_scores)
    inten, qual, topk, spars = _finalize(partials)
    return (attn_out, inten[0], qual[0], topk[0], spars[0])


# speedup vs baseline: 1.0612x; 1.0612x over previous
import functools

import jax
import jax.numpy as jnp
from jax import lax
from jax.experimental import pallas as pl
from jax.experimental.pallas import tpu as pltpu
from jax.experimental.pallas import tpu_sc as plsc

_ROWS = 64
_COLS = 32768
_N = _ROWS * _COLS
_NC = 2
_NS = 16
_NW = _NC * _NS
_PER_W = _N // _NW
_HALF = _PER_W // 2
_LANES = 16
_UNROLL = 4
_TOP_K = 64
_SEL_THR = 0.1


_ROWS_W = 8
_COLS_W = _COLS // 4
_COLS_H = _COLS_W // 2


def _sc_partials(attn):
    mesh = plsc.VectorSubcoreMesh(core_axis_name="c", subcore_axis_name="s")

    @functools.partial(
        pl.kernel,
        mesh=mesh,
        out_type=(
            jax.ShapeDtypeStruct((3 * _NW * _LANES,), jnp.float32),
            jax.ShapeDtypeStruct((_ROWS, _COLS), jnp.float32),
        ),
        compiler_params=pltpu.CompilerParams(
            use_tc_tiling_on_sc=True, needs_layout_passes=False),
        scratch_types=[
            pltpu.VMEM((2, _ROWS_W, _COLS_H), jnp.float32),
            pltpu.VMEM((3, _LANES), jnp.float32),
            pltpu.SemaphoreType.DMA,
            pltpu.SemaphoreType.DMA,
            pltpu.SemaphoreType.DMA,
            pltpu.SemaphoreType.DMA,
        ],
    )
    def k(x_hbm, out_hbm, attn_hbm, buf, stage, sem0, sem1, sem2, sem3):
        wid = lax.axis_index("s") * _NC + lax.axis_index("c")
        r0 = (wid // 4) * _ROWS_W
        c0 = (wid % 4) * _COLS_W

        cp0 = pltpu.async_copy(
            x_hbm.at[pl.ds(r0, _ROWS_W), pl.ds(c0, _COLS_H)], buf.at[0], sem0)
        cp1 = pltpu.async_copy(
            x_hbm.at[pl.ds(r0, _ROWS_W), pl.ds(c0 + _COLS_H, _COLS_H)],
            buf.at[1], sem1)

        fzeros = jnp.zeros((_LANES,), jnp.float32)
        izeros = jnp.zeros((_LANES,), jnp.int32)
        init = (fzeros, fzeros, izeros) * _UNROLL
        group = _LANES * _UNROLL

        def make_half(h):
            def row_body(r, carry):
                def body(i, accs_in):
                    accs = list(accs_in)
                    for j in range(_UNROLL):
                        v = buf[h, r, pl.ds(i + j * _LANES, _LANES)]
                        accs[3 * j] = accs[3 * j] + v
                        accs[3 * j + 1] = accs[3 * j + 1] + v * v
                        accs[3 * j + 2] = accs[3 * j + 2] + (
                            plsc.all_reduce_population_count(v > _SEL_THR))
                    return tuple(accs)
                return plsc.parallel_loop(
                    0, _COLS_H, step=group, unroll=2, carry=carry)(body)
            return row_body

        cp0.wait()
        wr0 = pltpu.async_copy(
            buf.at[0], attn_hbm.at[pl.ds(r0, _ROWS_W), pl.ds(c0, _COLS_H)],
            sem2)
        res = lax.fori_loop(0, _ROWS_W, make_half(0), init)
        cp1.wait()
        wr1 = pltpu.async_copy(
            buf.at[1],
            attn_hbm.at[pl.ds(r0, _ROWS_W), pl.ds(c0 + _COLS_H, _COLS_H)],
            sem3)
        res = lax.fori_loop(0, _ROWS_W, make_half(1), res)
        wr0.wait()
        wr1.wait()

        def tree_sum(vals):
            vals = list(vals)
            while len(vals) > 1:
                vals = [a + b for a, b in zip(vals[::2], vals[1::2])]
            return vals[0]

        s = tree_sum(res[0::3])
        q = tree_sum(res[1::3])
        c = tree_sum(res[2::3])
        stage[0] = s
        stage[1] = q
        stage[2] = c.astype(jnp.float32) * (1.0 / _LANES)
        for j in range(3):
            pltpu.sync_copy(
                stage.at[j],
                out_hbm.at[pl.ds((j * _NW + wid) * _LANES, _LANES)])

    return k(attn)


def _finalize(partials):
    def body(p_ref, inten_ref, qual_ref, topk_ref, spars_ref):
        w = _NW * _LANES
        s = jnp.sum(p_ref[pl.ds(0, w)])
        sq = jnp.sum(p_ref[pl.ds(w, w)])
        c = jnp.sum(p_ref[pl.ds(2 * w, w)])
        n = jnp.float32(_N)
        mean = s / n
        var = (sq - n * mean * mean) / (n - 1.0)
        std = jnp.sqrt(jnp.maximum(var, 0.0))
        consistency = 1.0 / (1.0 + std)
        focus = jnp.maximum(0.0, mean - 0.5) * 2.0
        inten_ref[0] = mean
        qual_ref[0] = jnp.clip(consistency - focus * 0.1, 0.0, 1.0)
        topk_ref[0] = jnp.minimum(c, jnp.float32(_TOP_K)).astype(jnp.int32)
        spars_ref[0] = 1.0 - c / n

    return pl.pallas_call(
        body,
        out_shape=(
            jax.ShapeDtypeStruct((1,), jnp.float32),
            jax.ShapeDtypeStruct((1,), jnp.float32),
            jax.ShapeDtypeStruct((1,), jnp.int32),
            jax.ShapeDtypeStruct((1,), jnp.float32),
        ),
        out_specs=(pl.BlockSpec(memory_space=pltpu.SMEM),) * 4,
    )(partials)


def kernel(attention_scores, features):
    partials, attn_out = _sc_partials(attention_scores)
    inten, qual, topk, spars = _finalize(partials)
    return (attn_out, inten[0], qual[0], topk[0], spars[0])
